# baseline (speedup 1.0000x reference)
import functools

import jax
import jax.numpy as jnp
from jax import lax
from jax.experimental import pallas as pl
from jax.experimental.pallas import tpu as pltpu
from jax.experimental.pallas import tpu_sc as plsc

HIDDEN = 128
HEADS = 4
T = 1
N_NODES = 10000
B = 64

N_TOT = B * T + N_NODES
N_TAB = 10112
DUMMY = N_TOT
RPT = N_TAB // 16
E_REAL = 320000 + N_NODES + N_NODES + N_TOT
CHUNK = 64
EPT = 11008
NCHUNK = EPT // CHUNK
E_PAD = EPT * 32
WCOL = 144
BN = 1264
GRID = N_TAB // BN


def _mm_bias_body(x_ref, w_ref, b_ref, o_ref):
    o_ref[...] = jnp.dot(x_ref[...], w_ref[...],
                         preferred_element_type=jnp.float32) + b_ref[...]


def _mm_bias(x, w, b, bn):
    r, k = x.shape
    m = w.shape[1]
    return pl.pallas_call(
        _mm_bias_body,
        grid=(r // bn,),
        in_specs=[
            pl.BlockSpec((bn, k), lambda i: (i, 0)),
            pl.BlockSpec((k, m), lambda i: (0, 0)),
            pl.BlockSpec((1, m), lambda i: (0, 0)),
        ],
        out_specs=pl.BlockSpec((bn, m), lambda i: (i, 0)),
        out_shape=jax.ShapeDtypeStruct((r, m), jnp.float32),
    )(x, w, b)


def _tc1_body(h_ref, w_ref, ss_ref, sd_ref, xp_ref, ts_ref, td_ref):
    xp = jnp.dot(h_ref[...], w_ref[...], preferred_element_type=jnp.float32)
    xp_ref[...] = xp
    ts_ref[...] = jnp.dot(xp, ss_ref[...], preferred_element_type=jnp.float32)
    td_ref[...] = jnp.dot(xp, sd_ref[...], preferred_element_type=jnp.float32)


def _tc1(h, w, s_src, s_dst):
    return pl.pallas_call(
        _tc1_body,
        grid=(GRID,),
        in_specs=[
            pl.BlockSpec((BN, HIDDEN), lambda i: (i, 0)),
            pl.BlockSpec((HIDDEN, HIDDEN), lambda i: (0, 0)),
            pl.BlockSpec((HIDDEN, 16), lambda i: (0, 0)),
            pl.BlockSpec((HIDDEN, 16), lambda i: (0, 0)),
        ],
        out_specs=[
            pl.BlockSpec((BN, HIDDEN), lambda i: (i, 0)),
            pl.BlockSpec((BN, 16), lambda i: (i, 0)),
            pl.BlockSpec((BN, 16), lambda i: (i, 0)),
        ],
        out_shape=[
            jax.ShapeDtypeStruct((N_TAB, HIDDEN), jnp.float32),
            jax.ShapeDtypeStruct((N_TAB, 16), jnp.float32),
            jax.ShapeDtypeStruct((N_TAB, 16), jnp.float32),
        ],
    )(h, w, s_src, s_dst)


def _tc2_body(h_ref, agg_ref, r_ref, gb_ref, lw_ref, lb_ref, o_ref):
    full = agg_ref[0] + agg_ref[1]
    agg = full[:, :HIDDEN]
    dexp = jnp.dot(full, r_ref[...], preferred_element_type=jnp.float32)
    hn = h_ref[...] + agg / dexp + gb_ref[...]
    mu = hn.mean(-1, keepdims=True)
    var = ((hn - mu) ** 2).mean(-1, keepdims=True)
    o_ref[...] = (hn - mu) / jnp.sqrt(var + 1e-5) * lw_ref[...] + lb_ref[...]


def _tc2(h, agg, r_sel, gat_b, ln_w, ln_b):
    return pl.pallas_call(
        _tc2_body,
        grid=(GRID,),
        in_specs=[
            pl.BlockSpec((BN, HIDDEN), lambda i: (i, 0)),
            pl.BlockSpec((2, BN, WCOL), lambda i: (0, i, 0)),
            pl.BlockSpec((WCOL, HIDDEN), lambda i: (0, 0)),
            pl.BlockSpec((1, HIDDEN), lambda i: (0, 0)),
            pl.BlockSpec((1, HIDDEN), lambda i: (0, 0)),
            pl.BlockSpec((1, HIDDEN), lambda i: (0, 0)),
        ],
        out_specs=pl.BlockSpec((BN, HIDDEN), lambda i: (i, 0)),
        out_shape=jax.ShapeDtypeStruct((N_TAB, HIDDEN), jnp.float32),
    )(h, agg, r_sel, gat_b, ln_w, ln_b)


_MESH = plsc.VectorSubcoreMesh(core_axis_name="c", subcore_axis_name="s")


@functools.partial(
    pl.kernel,
    mesh=_MESH,
    compiler_params=pltpu.CompilerParams(use_tc_tiling_on_sc=False),
    out_type=[jax.ShapeDtypeStruct((2, N_TAB, WCOL), jnp.float32)],
    scratch_types=[
        pltpu.VMEM((2, CHUNK, HIDDEN), jnp.float32),
        pltpu.VMEM((2, CHUNK, WCOL), jnp.float32),
        pltpu.VMEM((2, CHUNK, 16), jnp.float32),
        pltpu.VMEM((2, CHUNK, 16), jnp.float32),
        pltpu.VMEM((4, CHUNK), jnp.int32),
        pltpu.VMEM((4, CHUNK), jnp.int32),
        pltpu.VMEM_SHARED((N_TAB, WCOL), jnp.float32),
        pltpu.SemaphoreType.DMA,
        pltpu.SemaphoreType.DMA,
        pltpu.SemaphoreType.DMA,
        pltpu.SemaphoreType.DMA,
        pltpu.SemaphoreType.DMA,
        pltpu.SemaphoreType.DMA,
    ],
)
def _sc_gat(xp_hbm, ts_hbm, td_hbm, src_hbm, dst_hbm, zw_hbm,
            out_hbm,
            rowbuf, wbuf, sbuf, dbuf, src_sb, dst_sb, out_sh,
            gsem0, gsem1, ssem0, ssem1, isem0, isem1):
    c = lax.axis_index("c")
    s = lax.axis_index("s")
    w = s * 2 + c
    row0 = s * RPT
    trow0 = w * NCHUNK
    pltpu.sync_copy(zw_hbm.at[pl.ds(row0, RPT)], out_sh.at[pl.ds(row0, RPT)])
    plsc.subcore_barrier()

    lanes = jnp.arange(16, dtype=jnp.int32)
    gsems = (gsem0, gsem1)
    ssems = (ssem0, ssem1)
    isems = (isem0, isem1)

    def idx_prefetch(j, b):
        slot = j & 3
        pltpu.async_copy(src_hbm.at[pl.ds(trow0 + j, 1)],
                         src_sb.at[pl.ds(slot, 1)], isems[b])
        pltpu.async_copy(dst_hbm.at[pl.ds(trow0 + j, 1)],
                         dst_sb.at[pl.ds(slot, 1)], isems[b])

    def idx_drain(j, b):
        slot = j & 3
        pltpu.make_async_copy(src_hbm.at[pl.ds(trow0 + j, 1)],
                              src_sb.at[pl.ds(slot, 1)], isems[b]).wait()
        pltpu.make_async_copy(dst_hbm.at[pl.ds(trow0 + j, 1)],
                              dst_sb.at[pl.ds(slot, 1)], isems[b]).wait()

    def gather_issue(j, b):
        slot = j & 3
        pltpu.async_copy(xp_hbm.at[src_sb.at[slot]], rowbuf.at[b], gsems[b])
        pltpu.async_copy(ts_hbm.at[src_sb.at[slot]], sbuf.at[b], gsems[b])
        pltpu.async_copy(td_hbm.at[dst_sb.at[slot]], dbuf.at[b], gsems[b])

    def gather_drain(j, b):
        slot = j & 3
        pltpu.make_async_copy(xp_hbm.at[src_sb.at[slot]], rowbuf.at[b],
                              gsems[b]).wait()
        pltpu.make_async_copy(ts_hbm.at[src_sb.at[slot]], sbuf.at[b],
                              gsems[b]).wait()
        pltpu.make_async_copy(td_hbm.at[dst_sb.at[slot]], dbuf.at[b],
                              gsems[b]).wait()

    def scatter_drain(j, b):
        slot = j & 3
        pltpu.make_async_copy(wbuf.at[b], out_sh.at[dst_sb.at[slot]],
                              ssems[b]).wait()

    def process(j, b):
        rb, wb, sb, db = rowbuf.at[b], wbuf.at[b], sbuf.at[b], dbuf.at[b]

        @plsc.parallel_loop(0, CHUNK, step=1, unroll=8)
        def edge_body(i):
            a = sb[i, pl.ds(0, 16)] + db[i, pl.ds(0, 16)]
            a = jnp.maximum(a, 0.0) + 0.2 * jnp.minimum(a, 0.0)
            e16 = jnp.exp(a)
            wb[i, pl.ds(HIDDEN, 16)] = jnp.where(lanes < 4, e16, 0.0)
            for h in range(HEADS):
                ev = jnp.full((16,), e16[h], dtype=jnp.float32)
                for q in range(2):
                    col = h * 32 + q * 16
                    wb[i, pl.ds(col, 16)] = rb[i, pl.ds(col, 16)] * ev
        pltpu.async_copy(wbuf.at[b], out_sh.at[dst_sb.at[j & 3]], ssems[b],
                         add=True)

    pltpu.sync_copy(src_hbm.at[pl.ds(trow0, 4)], src_sb)
    pltpu.sync_copy(dst_hbm.at[pl.ds(trow0, 4)], dst_sb)
    gather_issue(0, 0)

    def outer_body(j0, carry):
        for b in range(2):
            j = j0 * 2 + b
            @pl.when(j0 > 0)
            def _():
                scatter_drain(j - 2, b)
            @pl.when(j0 < NCHUNK // 2 - 1)
            def _():
                idx_prefetch(j + 2, b)
            if b == 0:
                @pl.when(j0 > 0)
                def _():
                    idx_drain(j + 1, 1)
                gather_issue(j + 1, 1)
            else:
                @pl.when(j0 < NCHUNK // 2 - 1)
                def _():
                    idx_drain(j + 1, 0)
                    gather_issue(j + 1, 0)
            gather_drain(j, b)
            process(j, b)
        return carry

    lax.fori_loop(0, NCHUNK // 2, outer_body, 0)
    scatter_drain(NCHUNK - 2, 0)
    scatter_drain(NCHUNK - 1, 1)
    plsc.subcore_barrier()
    pltpu.sync_copy(out_sh.at[pl.ds(row0, RPT)],
                    out_hbm.at[c, pl.ds(row0, RPT)])


def kernel(x, edge_index, batch, W_in, b_in, summary_tokens, gat_W, att_src,
           att_dst, gat_b, ln_w, ln_b, W_out, b_out):
    offset = B * T

    node_idx = jnp.arange(N_NODES, dtype=jnp.int32) + offset
    loops = jnp.arange(N_TOT, dtype=jnp.int32)
    s_idx = (batch * T).astype(jnp.int32)
    src = jnp.concatenate([edge_index[0].astype(jnp.int32) + offset, s_idx,
                           node_idx, loops])
    dst = jnp.concatenate([edge_index[1].astype(jnp.int32) + offset, node_idx,
                           s_idx, loops])
    pad = jnp.full((E_PAD - E_REAL,), DUMMY, jnp.int32)
    src = jnp.concatenate([src, pad]).reshape(E_PAD // CHUNK, CHUNK)
    dst = jnp.concatenate([dst, pad]).reshape(E_PAD // CHUNK, CHUNK)

    x_pad = jnp.pad(x, ((0, N_TAB - N_NODES), (0, 0)))
    h0 = _mm_bias(x_pad, W_in, b_in[None, :], BN)[:N_NODES]
    h = jnp.concatenate([
        jnp.tile(summary_tokens, (B, 1)), h0,
        jnp.zeros((N_TAB - N_TOT, HIDDEN), jnp.float32)])

    def _sel(a):
        m = (jnp.eye(HEADS, dtype=jnp.float32)[:, None, :]
             * a[:, :, None]).reshape(HIDDEN, HEADS)
        return jnp.pad(m, ((0, 0), (0, 12)))

    r_sel = jnp.zeros((WCOL, HIDDEN), jnp.float32)
    r_sel = r_sel.at[HIDDEN:HIDDEN + HEADS, :].set(
        jnp.repeat(jnp.eye(HEADS, dtype=jnp.float32), HIDDEN // HEADS, axis=1))
    zw = jnp.zeros((N_TAB, WCOL), jnp.float32)

    for l in range(3):
        xp, ts, td = _tc1(h, gat_W[l], _sel(att_src[l]), _sel(att_dst[l]))
        (agg,) = _sc_gat(xp, ts, td, src, dst, zw)
        h = _tc2(h, agg, r_sel, gat_b[l][None, :], ln_w[l][None, :],
                 ln_b[l][None, :])

    proj = _mm_bias(h[:B * T], W_out, b_out[None, :], B * T)
    return proj.reshape(B, T, HIDDEN)


# speedup vs baseline: 1.1011x; 1.1011x over previous
import functools

import jax
import jax.numpy as jnp
from jax import lax
from jax.experimental import pallas as pl
from jax.experimental.pallas import tpu as pltpu
from jax.experimental.pallas import tpu_sc as plsc

HIDDEN = 128
HEADS = 4
T = 1
N_NODES = 10000
B = 64

N_TOT = B * T + N_NODES
N_TAB = 10112
DUMMY = N_TOT
RPT = N_TAB // 16
E_REAL = 320000 + N_NODES + N_NODES + N_TOT
CHUNK = 64
EPT = 11008
NCHUNK = EPT // CHUNK
E_PAD = EPT * 32
WCOL = 144
BN = 1264
GRID = N_TAB // BN


def _mm_bias_body(x_ref, w_ref, b_ref, o_ref):
    o_ref[...] = jnp.dot(x_ref[...], w_ref[...],
                         preferred_element_type=jnp.float32) + b_ref[...]


def _mm_bias(x, w, b, bn):
    r, k = x.shape
    m = w.shape[1]
    return pl.pallas_call(
        _mm_bias_body,
        grid=(r // bn,),
        in_specs=[
            pl.BlockSpec((bn, k), lambda i: (i, 0)),
            pl.BlockSpec((k, m), lambda i: (0, 0)),
            pl.BlockSpec((1, m), lambda i: (0, 0)),
        ],
        out_specs=pl.BlockSpec((bn, m), lambda i: (i, 0)),
        out_shape=jax.ShapeDtypeStruct((r, m), jnp.float32),
    )(x, w, b)


def _tc1_body(h_ref, w_ref, ss_ref, sd_ref, xps_ref, td_ref):
    xp = jnp.dot(h_ref[...], w_ref[...], preferred_element_type=jnp.float32)
    ts = jnp.dot(xp, ss_ref[...], preferred_element_type=jnp.float32)
    xps_ref[...] = jnp.concatenate([xp, ts], axis=1)
    td_ref[...] = jnp.dot(xp, sd_ref[...], preferred_element_type=jnp.float32)


def _tc1(h, w, s_src, s_dst):
    return pl.pallas_call(
        _tc1_body,
        grid=(GRID,),
        in_specs=[
            pl.BlockSpec((BN, HIDDEN), lambda i: (i, 0)),
            pl.BlockSpec((HIDDEN, HIDDEN), lambda i: (0, 0)),
            pl.BlockSpec((HIDDEN, 16), lambda i: (0, 0)),
            pl.BlockSpec((HIDDEN, 16), lambda i: (0, 0)),
        ],
        out_specs=[
            pl.BlockSpec((BN, HIDDEN + 16), lambda i: (i, 0)),
            pl.BlockSpec((BN, 16), lambda i: (i, 0)),
        ],
        out_shape=[
            jax.ShapeDtypeStruct((N_TAB, HIDDEN + 16), jnp.float32),
            jax.ShapeDtypeStruct((N_TAB, 16), jnp.float32),
        ],
    )(h, w, s_src, s_dst)


def _tc2_body(h_ref, agg_ref, r_ref, gb_ref, lw_ref, lb_ref, o_ref):
    full = agg_ref[0] + agg_ref[1]
    agg = full[:, :HIDDEN]
    dexp = jnp.dot(full, r_ref[...], preferred_element_type=jnp.float32)
    hn = h_ref[...] + agg / dexp + gb_ref[...]
    mu = hn.mean(-1, keepdims=True)
    var = ((hn - mu) ** 2).mean(-1, keepdims=True)
    o_ref[...] = (hn - mu) / jnp.sqrt(var + 1e-5) * lw_ref[...] + lb_ref[...]


def _tc2(h, agg, r_sel, gat_b, ln_w, ln_b):
    return pl.pallas_call(
        _tc2_body,
        grid=(GRID,),
        in_specs=[
            pl.BlockSpec((BN, HIDDEN), lambda i: (i, 0)),
            pl.BlockSpec((2, BN, WCOL), lambda i: (0, i, 0)),
            pl.BlockSpec((WCOL, HIDDEN), lambda i: (0, 0)),
            pl.BlockSpec((1, HIDDEN), lambda i: (0, 0)),
            pl.BlockSpec((1, HIDDEN), lambda i: (0, 0)),
            pl.BlockSpec((1, HIDDEN), lambda i: (0, 0)),
        ],
        out_specs=pl.BlockSpec((BN, HIDDEN), lambda i: (i, 0)),
        out_shape=jax.ShapeDtypeStruct((N_TAB, HIDDEN), jnp.float32),
    )(h, agg, r_sel, gat_b, ln_w, ln_b)


_MESH = plsc.VectorSubcoreMesh(core_axis_name="c", subcore_axis_name="s")


@functools.partial(
    pl.kernel,
    mesh=_MESH,
    compiler_params=pltpu.CompilerParams(use_tc_tiling_on_sc=False),
    out_type=[jax.ShapeDtypeStruct((2, N_TAB, WCOL), jnp.float32)],
    scratch_types=[
        pltpu.VMEM((2, CHUNK, HIDDEN + 16), jnp.float32),
        pltpu.VMEM((2, CHUNK, WCOL), jnp.float32),
        pltpu.VMEM((2, CHUNK, 16), jnp.float32),
        pltpu.VMEM((4, CHUNK), jnp.int32),
        pltpu.VMEM((4, CHUNK), jnp.int32),
        pltpu.VMEM_SHARED((N_TAB, WCOL), jnp.float32),
        pltpu.SemaphoreType.DMA,
        pltpu.SemaphoreType.DMA,
        pltpu.SemaphoreType.DMA,
        pltpu.SemaphoreType.DMA,
        pltpu.SemaphoreType.DMA,
        pltpu.SemaphoreType.DMA,
    ],
)
def _sc_gat(xp_hbm, td_hbm, src_hbm, dst_hbm, zw_hbm,
            out_hbm,
            rowbuf, wbuf, dbuf, src_sb, dst_sb, out_sh,
            gsem0, gsem1, ssem0, ssem1, isem0, isem1):
    c = lax.axis_index("c")
    s = lax.axis_index("s")
    w = s * 2 + c
    row0 = s * RPT
    trow0 = w * NCHUNK
    pltpu.sync_copy(zw_hbm.at[pl.ds(row0, RPT)], out_sh.at[pl.ds(row0, RPT)])
    plsc.subcore_barrier()

    lanes = jnp.arange(16, dtype=jnp.int32)
    gsems = (gsem0, gsem1)
    ssems = (ssem0, ssem1)
    isems = (isem0, isem1)

    def idx_prefetch(j, b):
        slot = j & 3
        pltpu.async_copy(src_hbm.at[pl.ds(trow0 + j, 1)],
                         src_sb.at[pl.ds(slot, 1)], isems[b])
        pltpu.async_copy(dst_hbm.at[pl.ds(trow0 + j, 1)],
                         dst_sb.at[pl.ds(slot, 1)], isems[b])

    def idx_drain(j, b):
        slot = j & 3
        pltpu.make_async_copy(src_hbm.at[pl.ds(trow0 + j, 1)],
                              src_sb.at[pl.ds(slot, 1)], isems[b]).wait()
        pltpu.make_async_copy(dst_hbm.at[pl.ds(trow0 + j, 1)],
                              dst_sb.at[pl.ds(slot, 1)], isems[b]).wait()

    def gather_issue(j, b):
        slot = j & 3
        pltpu.async_copy(xp_hbm.at[src_sb.at[slot]], rowbuf.at[b], gsems[b])
        pltpu.async_copy(td_hbm.at[dst_sb.at[slot]], dbuf.at[b], gsems[b])

    def gather_drain(j, b):
        slot = j & 3
        pltpu.make_async_copy(xp_hbm.at[src_sb.at[slot]], rowbuf.at[b],
                              gsems[b]).wait()
        pltpu.make_async_copy(td_hbm.at[dst_sb.at[slot]], dbuf.at[b],
                              gsems[b]).wait()

    def scatter_drain(j, b):
        slot = j & 3
        pltpu.make_async_copy(wbuf.at[b], out_sh.at[dst_sb.at[slot]],
                              ssems[b]).wait()

    def process(j, b):
        rb, wb, db = rowbuf.at[b], wbuf.at[b], dbuf.at[b]

        @plsc.parallel_loop(0, CHUNK, step=1, unroll=8)
        def edge_body(i):
            a = rb[i, pl.ds(HIDDEN, 16)] + db[i, pl.ds(0, 16)]
            a = jnp.maximum(a, 0.0) + 0.2 * jnp.minimum(a, 0.0)
            e16 = jnp.exp(a)
            wb[i, pl.ds(HIDDEN, 16)] = jnp.where(lanes < 4, e16, 0.0)
            for h in range(HEADS):
                ev = jnp.full((16,), e16[h], dtype=jnp.float32)
                for q in range(2):
                    col = h * 32 + q * 16
                    wb[i, pl.ds(col, 16)] = rb[i, pl.ds(col, 16)] * ev
        pltpu.async_copy(wbuf.at[b], out_sh.at[dst_sb.at[j & 3]], ssems[b],
                         add=True)

    pltpu.sync_copy(src_hbm.at[pl.ds(trow0, 4)], src_sb)
    pltpu.sync_copy(dst_hbm.at[pl.ds(trow0, 4)], dst_sb)
    gather_issue(0, 0)

    def outer_body(j0, carry):
        for b in range(2):
            j = j0 * 2 + b
            @pl.when(j0 > 0)
            def _():
                scatter_drain(j - 2, b)
            @pl.when(j0 < NCHUNK // 2 - 1)
            def _():
                idx_prefetch(j + 2, b)
            if b == 0:
                @pl.when(j0 > 0)
                def _():
                    idx_drain(j + 1, 1)
                gather_issue(j + 1, 1)
            else:
                @pl.when(j0 < NCHUNK // 2 - 1)
                def _():
                    idx_drain(j + 1, 0)
                    gather_issue(j + 1, 0)
            gather_drain(j, b)
            process(j, b)
        return carry

    lax.fori_loop(0, NCHUNK // 2, outer_body, 0)
    scatter_drain(NCHUNK - 2, 0)
    scatter_drain(NCHUNK - 1, 1)
    plsc.subcore_barrier()
    pltpu.sync_copy(out_sh.at[pl.ds(row0, RPT)],
                    out_hbm.at[c, pl.ds(row0, RPT)])


def kernel(x, edge_index, batch, W_in, b_in, summary_tokens, gat_W, att_src,
           att_dst, gat_b, ln_w, ln_b, W_out, b_out):
    offset = B * T

    node_idx = jnp.arange(N_NODES, dtype=jnp.int32) + offset
    loops = jnp.arange(N_TOT, dtype=jnp.int32)
    s_idx = (batch * T).astype(jnp.int32)
    src = jnp.concatenate([edge_index[0].astype(jnp.int32) + offset, s_idx,
                           node_idx, loops])
    dst = jnp.concatenate([edge_index[1].astype(jnp.int32) + offset, node_idx,
                           s_idx, loops])
    pad = jnp.full((E_PAD - E_REAL,), DUMMY, jnp.int32)
    src = jnp.concatenate([src, pad]).reshape(E_PAD // CHUNK, CHUNK)
    dst = jnp.concatenate([dst, pad]).reshape(E_PAD // CHUNK, CHUNK)

    x_pad = jnp.pad(x, ((0, N_TAB - N_NODES), (0, 0)))
    h0 = _mm_bias(x_pad, W_in, b_in[None, :], BN)[:N_NODES]
    h = jnp.concatenate([
        jnp.tile(summary_tokens, (B, 1)), h0,
        jnp.zeros((N_TAB - N_TOT, HIDDEN), jnp.float32)])

    def _sel(a):
        m = (jnp.eye(HEADS, dtype=jnp.float32)[:, None, :]
             * a[:, :, None]).reshape(HIDDEN, HEADS)
        return jnp.pad(m, ((0, 0), (0, 12)))

    r_sel = jnp.zeros((WCOL, HIDDEN), jnp.float32)
    r_sel = r_sel.at[HIDDEN:HIDDEN + HEADS, :].set(
        jnp.repeat(jnp.eye(HEADS, dtype=jnp.float32), HIDDEN // HEADS, axis=1))
    zw = jnp.zeros((N_TAB, WCOL), jnp.float32)

    for l in range(3):
        xps, td = _tc1(h, gat_W[l], _sel(att_src[l]), _sel(att_dst[l]))
        (agg,) = _sc_gat(xps, td, src, dst, zw)
        h = _tc2(h, agg, r_sel, gat_b[l][None, :], ln_w[l][None, :],
                 ln_b[l][None, :])

    proj = _mm_bias(h[:B * T], W_out, b_out[None, :], B * T)
    return proj.reshape(B, T, HIDDEN)
